# baseline (device time: 96835 ns/iter reference)
import jax
import jax.numpy as jnp
from jax import lax
from jax.experimental import pallas as pl
from jax.experimental.pallas import tpu as pltpu

N_DEV = 4
E_LOCAL = 4
N_EXPERTS = 16
N_TOK = 1024
D_MODEL = 512
D_HID = 1024
ROWS_Q = N_TOK // N_DEV


def kernel(x, router_W, route_idx, expert_W, shared_W):
    def body(x_ref, rw_ref, idx_ref, ew_ref, sw_ref, out_ref,
             comm_ref, send_sems, recv_sems):
        my_pos = lax.axis_index("i")
        right = lax.rem(my_pos + 1, N_DEV)

        xf = x_ref[:, :]
        scores = jnp.dot(xf, rw_ref[:, :],
                         preferred_element_type=jnp.float32)
        m = jnp.max(scores, axis=1, keepdims=True)
        p = jnp.exp(scores - m)
        p = p / jnp.sum(p, axis=1, keepdims=True)
        idx = idx_ref[:, :]
        onehot = idx == lax.broadcasted_iota(jnp.int32, (N_TOK, N_EXPERTS), 1)
        p_chosen = jnp.sum(jnp.where(onehot, p, 0.0), axis=1, keepdims=True)

        local_ids = my_pos * E_LOCAL + lax.broadcasted_iota(
            jnp.int32, (N_TOK, E_LOCAL), 1)
        w4 = jnp.where(idx == local_ids, p_chosen, 0.0)

        acc = jnp.zeros((N_TOK, D_HID), jnp.float32)
        for e in range(E_LOCAL):
            xe = (xf * w4[:, e:e + 1]).astype(jnp.bfloat16)
            acc = acc + jnp.dot(xe, ew_ref[e, :, :].astype(jnp.bfloat16),
                                preferred_element_type=jnp.float32)

        xq = x_ref[pl.ds(my_pos * ROWS_Q, ROWS_Q), :].astype(jnp.bfloat16)
        shared_q = jnp.dot(xq, sw_ref[:, :].astype(jnp.bfloat16),
                           preferred_element_type=jnp.float32)

        out_ref[:, :] = acc
        row0 = my_pos * ROWS_Q
        out_ref[pl.ds(row0, ROWS_Q), :] = (
            out_ref[pl.ds(row0, ROWS_Q), :] + shared_q)

        comm_ref[0, :, :] = out_ref[:, :].astype(jnp.bfloat16)
        for h in range(N_DEV - 1):
            rdma = pltpu.make_async_remote_copy(
                src_ref=comm_ref.at[h],
                dst_ref=comm_ref.at[h + 1],
                send_sem=send_sems.at[h],
                recv_sem=recv_sems.at[h],
                device_id=(right,),
                device_id_type=pl.DeviceIdType.MESH,
            )
            rdma.start()
            rdma.wait()
            out_ref[:, :] = out_ref[:, :] + comm_ref[h + 1].astype(jnp.float32)

    return pl.pallas_call(
        body,
        out_shape=jax.ShapeDtypeStruct((N_TOK, D_HID), jnp.float32),
        in_specs=[
            pl.BlockSpec(memory_space=pltpu.VMEM),
            pl.BlockSpec(memory_space=pltpu.VMEM),
            pl.BlockSpec(memory_space=pltpu.VMEM),
            pl.BlockSpec(memory_space=pltpu.VMEM),
            pl.BlockSpec(memory_space=pltpu.VMEM),
        ],
        out_specs=pl.BlockSpec(memory_space=pltpu.VMEM),
        scratch_shapes=[
            pltpu.VMEM((N_DEV, N_TOK, D_HID), jnp.bfloat16),
            pltpu.SemaphoreType.DMA((N_DEV - 1,)),
            pltpu.SemaphoreType.DMA((N_DEV - 1,)),
        ],
    )(x, router_W, route_idx, expert_W, shared_W)
